# baseline (speedup 1.0000x reference)
import jax
import jax.numpy as jnp
from jax.experimental import pallas as pl


def kernel(x, embed_table):
    b, _, h, w = x.shape
    hw = h * w
    d = embed_table.shape[1]

    bb = 4
    grid = b // bb

    def body(e_ref, o_ref):
        o_ref[...] = jnp.broadcast_to(e_ref[...][None], (bb, hw, d))

    out = pl.pallas_call(
        body,
        grid=(grid,),
        in_specs=[pl.BlockSpec((hw, d), lambda i: (0, 0))],
        out_specs=pl.BlockSpec((bb, hw, d), lambda i: (i, 0, 0)),
        out_shape=jax.ShapeDtypeStruct((b, hw, d), embed_table.dtype),
    )(embed_table)
    return out.reshape(b, h, w, d).transpose(0, 3, 1, 2)


# speedup vs baseline: 1.3337x; 1.3337x over previous
import jax
import jax.numpy as jnp
from jax.experimental import pallas as pl


def kernel(x, embed_table):
    b, _, h, w = x.shape
    hw = h * w
    d = embed_table.shape[1]

    bb = 16
    grid = b // bb

    def body(e_ref, o_ref):
        o_ref[...] = jnp.broadcast_to(e_ref[...][None], (bb, hw, d))

    out = pl.pallas_call(
        body,
        grid=(grid,),
        in_specs=[pl.BlockSpec((hw, d), lambda i: (0, 0))],
        out_specs=pl.BlockSpec((bb, hw, d), lambda i: (i, 0, 0)),
        out_shape=jax.ShapeDtypeStruct((b, hw, d), embed_table.dtype),
    )(embed_table)
    return out.reshape(b, h, w, d).transpose(0, 3, 1, 2)


# speedup vs baseline: 1.4016x; 1.0509x over previous
import jax
import jax.numpy as jnp
from jax.experimental import pallas as pl
from jax.experimental.pallas import tpu as pltpu


def kernel(x, embed_table):
    b, _, h, w = x.shape
    hw = h * w
    d = embed_table.shape[1]

    def body(e_ref, o_ref, sem):
        for bb in range(b):
            pltpu.make_async_copy(e_ref, o_ref.at[bb], sem).start()
        for bb in range(b):
            pltpu.make_async_copy(e_ref, o_ref.at[bb], sem).wait()

    out = pl.pallas_call(
        body,
        in_specs=[pl.BlockSpec(memory_space=pltpu.MemorySpace.VMEM)],
        out_specs=pl.BlockSpec(memory_space=pltpu.MemorySpace.HBM),
        out_shape=jax.ShapeDtypeStruct((b, hw, d), embed_table.dtype),
        scratch_shapes=[pltpu.SemaphoreType.DMA],
    )(embed_table)
    return out.reshape(b, h, w, d).transpose(0, 3, 1, 2)
